# baseline (speedup 1.0000x reference)
import functools

import jax
import jax.numpy as jnp
from jax import lax
from jax.experimental import pallas as pl
from jax.experimental.pallas import tpu as pltpu
from jax.experimental.pallas import tpu_sc as plsc

SEQ = 4
NAA = 20
VOCAB = NAA**SEQ
B = 16384
F = SEQ * NAA
NC, NS, L = 2, 16, 16
NW = NC * NS
BPW = B // NW
NGRP = BPW // L
GCHUNK = 128
NGATHER = BPW // GCHUNK

_mesh = plsc.VectorSubcoreMesh(
    core_axis_name="c", subcore_axis_name="s", num_cores=NC, num_subcores=NS
)


@functools.partial(
    pl.kernel,
    out_type=jax.ShapeDtypeStruct((B,), jnp.float32),
    mesh=_mesh,
    compiler_params=pltpu.CompilerParams(needs_layout_passes=False),
    scratch_types=[
        pltpu.VMEM((BPW * F,), jnp.float32),
        pltpu.VMEM((F,), jnp.float32),
        pltpu.VMEM((BPW,), jnp.int32),
        pltpu.VMEM((BPW,), jnp.float32),
        pltpu.SemaphoreType.DMA,
    ],
)
def _sc_fwd(x_hbm, fit_hbm, mf_hbm, out_hbm, x_v, mf_v, idx_v, val_v, sem):
    wid = lax.axis_index("s") * NC + lax.axis_index("c")
    base = wid * BPW

    pltpu.sync_copy(mf_hbm, mf_v)
    pltpu.sync_copy(x_hbm.at[pl.ds(base * F, BPW * F)], x_v)

    lanes = lax.iota(jnp.int32, L)
    mf_vecs = [mf_v[pl.ds(k * L, L)] for k in range(F // L)]

    def group_body(g, carry):
        row0 = pl.multiple_of(g * L, L)
        flat0 = (row0 + lanes) * F
        acc = jnp.zeros((L,), jnp.float32)
        for j in range(F):
            v = plsc.load_gather(x_v, [flat0 + j])
            acc = acc + v * mf_vecs[j // L][j % L]
        idx16 = jnp.clip(acc, 0.0, float(VOCAB - 1)).astype(jnp.int32)
        idx_v[pl.ds(row0, L)] = idx16
        return carry

    lax.fori_loop(0, NGRP, group_body, 0)

    copies = [
        pltpu.async_copy(
            fit_hbm.at[idx_v.at[pl.ds(t * GCHUNK, GCHUNK)]],
            val_v.at[pl.ds(t * GCHUNK, GCHUNK)],
            sem,
        )
        for t in range(NGATHER)
    ]
    for c in copies:
        c.wait()

    pltpu.sync_copy(val_v, out_hbm.at[pl.ds(base, BPW)])


def kernel(x, fitnesses, mult_factor):
    x_flat = x.reshape(B * F)
    fit_flat = fitnesses.reshape(VOCAB)
    mf_flat = mult_factor.reshape(F)
    return _sc_fwd(x_flat, fit_flat, mf_flat)


# speedup vs baseline: 1.0565x; 1.0565x over previous
import functools

import jax
import jax.numpy as jnp
from jax import lax
from jax.experimental import pallas as pl
from jax.experimental.pallas import tpu as pltpu
from jax.experimental.pallas import tpu_sc as plsc

SEQ = 4
NAA = 20
VOCAB = NAA**SEQ
B = 16384
F = SEQ * NAA
NC, NS, L = 2, 16, 16
NW = NC * NS
BPW = B // NW
CHUNK = 128
NCHUNK = BPW // CHUNK
NGRP = CHUNK // L
NACC = 8

_mesh = plsc.VectorSubcoreMesh(
    core_axis_name="c", subcore_axis_name="s", num_cores=NC, num_subcores=NS
)


@functools.partial(
    pl.kernel,
    out_type=jax.ShapeDtypeStruct((B,), jnp.float32),
    mesh=_mesh,
    compiler_params=pltpu.CompilerParams(needs_layout_passes=False),
    scratch_types=[
        pltpu.VMEM((CHUNK * F,), jnp.float32),
        pltpu.VMEM((CHUNK * F,), jnp.float32),
        pltpu.VMEM((F,), jnp.float32),
        pltpu.VMEM((BPW,), jnp.int32),
        pltpu.VMEM((BPW,), jnp.float32),
        pltpu.SemaphoreType.DMA,
        pltpu.SemaphoreType.DMA,
        pltpu.SemaphoreType.DMA,
    ],
)
def _sc_fwd(x_hbm, fit_hbm, mf_hbm, out_hbm, x_v0, x_v1, mf_v, idx_v, val_v, s0, s1, sg):
    wid = lax.axis_index("s") * NC + lax.axis_index("c")
    base = wid * BPW

    pltpu.sync_copy(mf_hbm, mf_v)
    mf_scalars = []
    for k in range(F // L):
        vec = mf_v[pl.ds(k * L, L)]
        mf_scalars.extend(vec[l] for l in range(L))

    lanes = lax.iota(jnp.int32, L)
    xbufs = (x_v0, x_v1)
    xsem = (s0, s1)

    def start_fetch(c):
        return pltpu.async_copy(
            x_hbm.at[pl.ds((base + c * CHUNK) * F, CHUNK * F)],
            xbufs[c % 2],
            xsem[c % 2],
        )

    pending_x = start_fetch(0)
    gathers = []
    for c in range(NCHUNK):
        nxt = start_fetch(c + 1) if c + 1 < NCHUNK else None
        pending_x.wait()
        pending_x = nxt
        xbuf = xbufs[c % 2]

        def group_body(g, carry, _c=c, _xbuf=xbuf):
            row0 = pl.multiple_of(g * L, L)
            flat0 = (row0 + lanes) * F
            accs = [jnp.zeros((L,), jnp.float32) for _ in range(NACC)]
            for j in range(F):
                v = plsc.load_gather(_xbuf, [flat0 + j])
                accs[j % NACC] = accs[j % NACC] + v * mf_scalars[j]
            while len(accs) > 1:
                accs = [
                    accs[i] + accs[i + 1] if i + 1 < len(accs) else accs[i]
                    for i in range(0, len(accs), 2)
                ]
            idx16 = jnp.clip(accs[0], 0.0, float(VOCAB - 1)).astype(jnp.int32)
            idx_v[pl.ds(pl.multiple_of(_c * CHUNK + row0, L), L)] = idx16
            return carry

        lax.fori_loop(0, NGRP, group_body, 0)
        gathers.append(
            pltpu.async_copy(
                fit_hbm.at[idx_v.at[pl.ds(c * CHUNK, CHUNK)]],
                val_v.at[pl.ds(c * CHUNK, CHUNK)],
                sg,
            )
        )

    for gcopy in gathers:
        gcopy.wait()
    pltpu.sync_copy(val_v, out_hbm.at[pl.ds(base, BPW)])


def kernel(x, fitnesses, mult_factor):
    x_flat = x.reshape(B * F)
    fit_flat = fitnesses.reshape(VOCAB)
    mf_flat = mult_factor.reshape(F)
    return _sc_fwd(x_flat, fit_flat, mf_flat)
